# Initial kernel scaffold; baseline (speedup 1.0000x reference)
import jax
import jax.numpy as jnp
from jax.experimental import pallas as pl


def kernel(x, weights, biases):
    raise NotImplementedError("write your pallas kernel here")


# speedup vs baseline: 1.8858x; 1.8858x over previous
import functools

import jax
import jax.numpy as jnp
from jax.experimental import pallas as pl
from jax.experimental.pallas import tpu as pltpu

_EPS = 1e-5


def _setnorm_kernel(x_ref, w_ref, b_ref, o_ref):
    x = x_ref[0]
    n = x.shape[0] * x.shape[1]
    s1 = jnp.sum(x, keepdims=True)
    s2 = jnp.sum(x * x, keepdims=True)
    mean = s1 * (1.0 / n)
    var = s2 * (1.0 / n) - mean * mean
    inv = jax.lax.rsqrt(var + _EPS)
    scale = w_ref[...] * inv
    shift = b_ref[...] - mean * scale
    o_ref[0] = x * scale + shift


@jax.jit
def kernel(x, weights, biases):
    B, N, F = x.shape
    w2 = weights.reshape(1, F)
    b2 = biases.reshape(1, F)
    grid = (B,)
    return pl.pallas_call(
        _setnorm_kernel,
        grid=grid,
        in_specs=[
            pl.BlockSpec((1, N, F), lambda b: (b, 0, 0)),
            pl.BlockSpec((1, F), lambda b: (0, 0)),
            pl.BlockSpec((1, F), lambda b: (0, 0)),
        ],
        out_specs=pl.BlockSpec((1, N, F), lambda b: (b, 0, 0)),
        out_shape=jax.ShapeDtypeStruct((B, N, F), x.dtype),
        compiler_params=pltpu.CompilerParams(
            dimension_semantics=("parallel",),
            vmem_limit_bytes=52 * 1024 * 1024,
        ),
    )(x, w2, b2)


# speedup vs baseline: 1.9259x; 1.0213x over previous
import functools

import jax
import jax.numpy as jnp
from jax.experimental import pallas as pl
from jax.experimental.pallas import tpu as pltpu

_EPS = 1e-5


def _setnorm_kernel(x_ref, w_ref, b_ref, o_ref):
    N, F = x_ref.shape[1], x_ref.shape[2]
    n = N * F
    C = 8
    step = N // C
    s1 = jnp.zeros((1, F), jnp.float32)
    s2 = jnp.zeros((1, F), jnp.float32)
    for i in range(C):
        lo, hi = i * step, (i + 1) * step
        xs = x_ref[0, lo:hi, :]
        s1 = s1 + jnp.sum(xs, axis=0, keepdims=True)
        s2 = s2 + jnp.sum(xs * xs, axis=0, keepdims=True)
    s1 = jnp.sum(s1, keepdims=True)
    s2 = jnp.sum(s2, keepdims=True)
    mean = s1 * (1.0 / n)
    var = s2 * (1.0 / n) - mean * mean
    inv = jax.lax.rsqrt(var + _EPS)
    scale = w_ref[...] * inv
    shift = b_ref[...] - mean * scale
    for i in range(C):
        lo, hi = i * step, (i + 1) * step
        o_ref[0, lo:hi, :] = x_ref[0, lo:hi, :] * scale + shift


@jax.jit
def kernel(x, weights, biases):
    B, N, F = x.shape
    w2 = weights.reshape(1, F)
    b2 = biases.reshape(1, F)
    grid = (B,)
    return pl.pallas_call(
        _setnorm_kernel,
        grid=grid,
        in_specs=[
            pl.BlockSpec((1, N, F), lambda b: (b, 0, 0)),
            pl.BlockSpec((1, F), lambda b: (0, 0)),
            pl.BlockSpec((1, F), lambda b: (0, 0)),
        ],
        out_specs=pl.BlockSpec((1, N, F), lambda b: (b, 0, 0)),
        out_shape=jax.ShapeDtypeStruct((B, N, F), x.dtype),
        compiler_params=pltpu.CompilerParams(
            dimension_semantics=("parallel",),
            vmem_limit_bytes=52 * 1024 * 1024,
        ),
    )(x, w2, b2)
